# Initial kernel scaffold; baseline (speedup 1.0000x reference)
import jax
import jax.numpy as jnp
from jax.experimental import pallas as pl


def kernel(ijk, pF, M):
    raise NotImplementedError("write your pallas kernel here")


# speedup vs baseline: 2.7511x; 2.7511x over previous
import functools

import jax
import jax.numpy as jnp
from jax import lax
from jax.experimental import pallas as pl
from jax.experimental.pallas import tpu as pltpu
from jax.experimental.pallas import tpu_sc as plsc

_ALPHA = 0.001
_BETA = 0.001
_S = 3
_R = 16
_DPF = _R * (1 + 2 * _S)
_BATCH = 16384
_NC, _NS, _L = 2, 16, 16
_NW = _NC * _NS
_BPW = _BATCH // _NW
_NG = _BPW // _L


def _mf_body(i_hbm, j_hbm, k0_hbm, k1_hbm, k2_hbm, pF_hbm, M_hbm, out_hbm,
             idx_i, idx_j, idx_k0, idx_k1, idx_k2,
             pf_buf, mj_buf, mk0_buf, mk1_buf, mk2_buf, out_buf,
             s0, s1, s2, s3, s4):
    wid = lax.axis_index("s") * _NC + lax.axis_index("c")
    base = wid * _BPW

    pltpu.sync_copy(i_hbm.at[pl.ds(base, _BPW)], idx_i)
    pltpu.sync_copy(j_hbm.at[pl.ds(base, _BPW)], idx_j)
    pltpu.sync_copy(k0_hbm.at[pl.ds(base, _BPW)], idx_k0)
    pltpu.sync_copy(k1_hbm.at[pl.ds(base, _BPW)], idx_k1)
    pltpu.sync_copy(k2_hbm.at[pl.ds(base, _BPW)], idx_k2)

    cps = [
        pltpu.async_copy(pF_hbm.at[idx_i], pf_buf, s0),
        pltpu.async_copy(M_hbm.at[idx_j], mj_buf, s1),
        pltpu.async_copy(M_hbm.at[idx_k0], mk0_buf, s2),
        pltpu.async_copy(M_hbm.at[idx_k1], mk1_buf, s3),
        pltpu.async_copy(M_hbm.at[idx_k2], mk2_buf, s4),
    ]
    for cp in cps:
        cp.wait()

    lane = lax.iota(jnp.int32, _L)

    def group(g, carry):
        row = g * _L + lane

        def col(c):
            return jnp.full((_L,), c, dtype=jnp.int32)

        mj = [plsc.load_gather(mj_buf, [row, col(r)]) for r in range(_R)]
        ms = [
            plsc.load_gather(mk0_buf, [row, col(r)])
            + plsc.load_gather(mk1_buf, [row, col(r)])
            + plsc.load_gather(mk2_buf, [row, col(r)])
            for r in range(_R)
        ]

        accp = None
        a = [None, None, None]
        b = [None, None, None]
        for c in range(_DPF):
            pv = plsc.load_gather(pf_buf, [row, col(c)])
            if c < _R:
                term = pv * mj[c]
                accp = term if accp is None else accp + term
            elif c < (1 + _S) * _R:
                r, s = divmod(c - _R, _S)
                term = pv * mj[r]
                a[s] = term if a[s] is None else a[s] + term
            else:
                r, s = divmod(c - (1 + _S) * _R, _S)
                term = pv * ms[r]
                b[s] = term if b[s] is None else b[s] + term

        res = _ALPHA * accp + (_BETA * _BETA) * (
            a[0] * b[0] + a[1] * b[1] + a[2] * b[2])
        plsc.store_scatter(out_buf, [row], res)
        return carry

    lax.fori_loop(0, _NG, group, 0)

    pltpu.sync_copy(out_buf, out_hbm.at[pl.ds(base, _BPW)])


@functools.partial(jax.jit, static_argnums=())
def _mf_call(i, j, k0, k1, k2, pF, M):
    mesh = plsc.VectorSubcoreMesh(core_axis_name="c", subcore_axis_name="s")
    f = functools.partial(
        pl.kernel,
        mesh=mesh,
        out_type=jax.ShapeDtypeStruct((_BATCH,), jnp.float32),
        compiler_params=pltpu.CompilerParams(
            use_tc_tiling_on_sc=False, needs_layout_passes=False),
        scratch_types=[
            pltpu.VMEM((_BPW,), jnp.int32),
            pltpu.VMEM((_BPW,), jnp.int32),
            pltpu.VMEM((_BPW,), jnp.int32),
            pltpu.VMEM((_BPW,), jnp.int32),
            pltpu.VMEM((_BPW,), jnp.int32),
            pltpu.VMEM((_BPW, _DPF), jnp.float32),
            pltpu.VMEM((_BPW, _R), jnp.float32),
            pltpu.VMEM((_BPW, _R), jnp.float32),
            pltpu.VMEM((_BPW, _R), jnp.float32),
            pltpu.VMEM((_BPW, _R), jnp.float32),
            pltpu.VMEM((_BPW,), jnp.float32),
            pltpu.SemaphoreType.DMA,
            pltpu.SemaphoreType.DMA,
            pltpu.SemaphoreType.DMA,
            pltpu.SemaphoreType.DMA,
            pltpu.SemaphoreType.DMA,
        ],
    )(_mf_body)
    return f(i, j, k0, k1, k2, pF, M)


def kernel(ijk, pF, M):
    i = ijk[:, 0]
    j = ijk[:, 1]
    k0 = ijk[:, 2]
    k1 = ijk[:, 3]
    k2 = ijk[:, 4]
    return _mf_call(i, j, k0, k1, k2, pF, M)


# speedup vs baseline: 2.8487x; 1.0355x over previous
import functools

import jax
import jax.numpy as jnp
from jax import lax
from jax.experimental import pallas as pl
from jax.experimental.pallas import tpu as pltpu
from jax.experimental.pallas import tpu_sc as plsc

_ALPHA = 0.001
_BETA = 0.001
_S = 3
_R = 16
_DPF = _R * (1 + 2 * _S)
_BATCH = 16384
_NC, _NS, _L = 2, 16, 16
_NW = _NC * _NS
_BPW = _BATCH // _NW
_CHUNK = 64
_NCH = _BPW // _CHUNK
_NGC = _CHUNK // _L


def _mf_body(i_hbm, j_hbm, k0_hbm, k1_hbm, k2_hbm, pF_hbm, M8_hbm, out_hbm,
             idx_i, idx_j, idx_k0, idx_k1, idx_k2,
             gj, gk0, gk1, gk2,
             pf_buf, mj_buf, mk0_buf, mk1_buf, mk2_buf, out_buf,
             s0, s1, s2, s3, s4):
    wid = lax.axis_index("s") * _NC + lax.axis_index("c")
    base = wid * _BPW

    pltpu.sync_copy(i_hbm.at[pl.ds(base, _BPW)], idx_i)
    pltpu.sync_copy(j_hbm.at[pl.ds(base, _BPW)], idx_j)
    pltpu.sync_copy(k0_hbm.at[pl.ds(base, _BPW)], idx_k0)
    pltpu.sync_copy(k1_hbm.at[pl.ds(base, _BPW)], idx_k1)
    pltpu.sync_copy(k2_hbm.at[pl.ds(base, _BPW)], idx_k2)

    cp_pf = pltpu.async_copy(pF_hbm.at[idx_i], pf_buf, s0)

    lane = lax.iota(jnp.int32, _L)

    def chunk_body(ch, carry):
        cb = ch * _CHUNK
        for idx_src, gdst in ((idx_j, gj), (idx_k0, gk0),
                              (idx_k1, gk1), (idx_k2, gk2)):
            for v in range(_CHUNK // _L):
                off = pl.multiple_of(cb + v * _L, _L)
                gdst[pl.ds(v * _L, _L)] = (
                    idx_src[pl.ds(off, _L)] >> 3)
        cps = [
            pltpu.async_copy(M8_hbm.at[gj], mj_buf, s1),
            pltpu.async_copy(M8_hbm.at[gk0], mk0_buf, s2),
            pltpu.async_copy(M8_hbm.at[gk1], mk1_buf, s3),
            pltpu.async_copy(M8_hbm.at[gk2], mk2_buf, s4),
        ]
        for cp in cps:
            cp.wait()

        def group(grp, carry2):
            slot = grp * _L + lane
            grow = cb + slot
            goff = pl.multiple_of(cb + grp * _L, _L)
            jv = idx_j[pl.ds(goff, _L)]
            k0v = idx_k0[pl.ds(goff, _L)]
            k1v = idx_k1[pl.ds(goff, _L)]
            k2v = idx_k2[pl.ds(goff, _L)]
            joff = (jv & 7) << 4
            k0off = (k0v & 7) << 4
            k1off = (k1v & 7) << 4
            k2off = (k2v & 7) << 4

            mj = [plsc.load_gather(mj_buf, [slot, joff + r])
                  for r in range(_R)]
            ms = [
                plsc.load_gather(mk0_buf, [slot, k0off + r])
                + plsc.load_gather(mk1_buf, [slot, k1off + r])
                + plsc.load_gather(mk2_buf, [slot, k2off + r])
                for r in range(_R)
            ]

            def col(c):
                return jnp.full((_L,), c, dtype=jnp.int32)

            accp = None
            a = [None, None, None]
            b = [None, None, None]
            for c in range(_DPF):
                pv = plsc.load_gather(pf_buf, [grow, col(c)])
                if c < _R:
                    term = pv * mj[c]
                    accp = term if accp is None else accp + term
                elif c < (1 + _S) * _R:
                    r, s = divmod(c - _R, _S)
                    term = pv * mj[r]
                    a[s] = term if a[s] is None else a[s] + term
                else:
                    r, s = divmod(c - (1 + _S) * _R, _S)
                    term = pv * ms[r]
                    b[s] = term if b[s] is None else b[s] + term

            res = _ALPHA * accp + (_BETA * _BETA) * (
                a[0] * b[0] + a[1] * b[1] + a[2] * b[2])
            plsc.store_scatter(out_buf, [grow], res)
            return carry2

        lax.fori_loop(0, _NGC, group, 0)
        return carry

    cp_pf.wait()
    lax.fori_loop(0, _NCH, chunk_body, 0)

    pltpu.sync_copy(out_buf, out_hbm.at[pl.ds(base, _BPW)])


@jax.jit
def _mf_call(i, j, k0, k1, k2, pF128, M8):
    mesh = plsc.VectorSubcoreMesh(core_axis_name="c", subcore_axis_name="s")
    f = functools.partial(
        pl.kernel,
        mesh=mesh,
        out_type=jax.ShapeDtypeStruct((_BATCH,), jnp.float32),
        compiler_params=pltpu.CompilerParams(
            use_tc_tiling_on_sc=True, needs_layout_passes=False),
        scratch_types=[
            pltpu.VMEM((_BPW,), jnp.int32),
            pltpu.VMEM((_BPW,), jnp.int32),
            pltpu.VMEM((_BPW,), jnp.int32),
            pltpu.VMEM((_BPW,), jnp.int32),
            pltpu.VMEM((_BPW,), jnp.int32),
            pltpu.VMEM((_CHUNK,), jnp.int32),
            pltpu.VMEM((_CHUNK,), jnp.int32),
            pltpu.VMEM((_CHUNK,), jnp.int32),
            pltpu.VMEM((_CHUNK,), jnp.int32),
            pltpu.VMEM((_BPW, 128), jnp.float32),
            pltpu.VMEM((_CHUNK, 128), jnp.float32),
            pltpu.VMEM((_CHUNK, 128), jnp.float32),
            pltpu.VMEM((_CHUNK, 128), jnp.float32),
            pltpu.VMEM((_CHUNK, 128), jnp.float32),
            pltpu.VMEM((_BPW,), jnp.float32),
            pltpu.SemaphoreType.DMA,
            pltpu.SemaphoreType.DMA,
            pltpu.SemaphoreType.DMA,
            pltpu.SemaphoreType.DMA,
            pltpu.SemaphoreType.DMA,
        ],
    )(_mf_body)
    return f(i, j, k0, k1, k2, pF128, M8)


def kernel(ijk, pF, M):
    i = ijk[:, 0]
    j = ijk[:, 1]
    k0 = ijk[:, 2]
    k1 = ijk[:, 3]
    k2 = ijk[:, 4]
    pF128 = jnp.pad(pF, ((0, 0), (0, 128 - _DPF)))
    M8 = M.reshape(-1, 128)
    return _mf_call(i, j, k0, k1, k2, pF128, M8)


# speedup vs baseline: 4.2814x; 1.5029x over previous
import functools

import jax
import jax.numpy as jnp
from jax import lax
from jax.experimental import pallas as pl
from jax.experimental.pallas import tpu as pltpu
from jax.experimental.pallas import tpu_sc as plsc

_ALPHA = 0.001
_BETA = 0.001
_S = 3
_R = 16
_DPF = _R * (1 + 2 * _S)
_BATCH = 16384
_NC, _NS, _L = 2, 16, 16
_NW = _NC * _NS
_BPW = _BATCH // _NW
_CHUNK = 64
_NCH = _BPW // _CHUNK
_NGC = _CHUNK // _L


def _mf_body(i_hbm, j_hbm, k0_hbm, k1_hbm, k2_hbm, pF_hbm, M8_hbm, out_hbm,
             idx_i, idx_j, idx_k0, idx_k1, idx_k2,
             gj, gk0, gk1, gk2,
             pf_buf, mj_buf, mk0_buf, mk1_buf, mk2_buf, out_buf,
             s0, s1, s2, s3, s4):
    wid = lax.axis_index("s") * _NC + lax.axis_index("c")
    base = wid * _BPW

    pltpu.sync_copy(i_hbm.at[pl.ds(base, _BPW)], idx_i)
    pltpu.sync_copy(j_hbm.at[pl.ds(base, _BPW)], idx_j)
    pltpu.sync_copy(k0_hbm.at[pl.ds(base, _BPW)], idx_k0)
    pltpu.sync_copy(k1_hbm.at[pl.ds(base, _BPW)], idx_k1)
    pltpu.sync_copy(k2_hbm.at[pl.ds(base, _BPW)], idx_k2)

    cp_pf = pltpu.async_copy(pF_hbm.at[idx_i], pf_buf, s0)

    lane = lax.iota(jnp.int32, _L)

    def chunk_body(ch, carry):
        cb = ch * _CHUNK
        for idx_src, gdst in ((idx_j, gj), (idx_k0, gk0),
                              (idx_k1, gk1), (idx_k2, gk2)):
            for v in range(_CHUNK // _L):
                off = pl.multiple_of(cb + v * _L, _L)
                gdst[pl.ds(v * _L, _L)] = (
                    idx_src[pl.ds(off, _L)] >> 3)
        cps = [
            pltpu.async_copy(M8_hbm.at[gj], mj_buf, s1),
            pltpu.async_copy(M8_hbm.at[gk0], mk0_buf, s2),
            pltpu.async_copy(M8_hbm.at[gk1], mk1_buf, s3),
            pltpu.async_copy(M8_hbm.at[gk2], mk2_buf, s4),
        ]
        for cp in cps:
            cp.wait()

        def group(grp, carry2):
            slot = grp * _L + lane
            grow = cb + slot
            goff = pl.multiple_of(cb + grp * _L, _L)
            jv = idx_j[pl.ds(goff, _L)]
            k0v = idx_k0[pl.ds(goff, _L)]
            k1v = idx_k1[pl.ds(goff, _L)]
            k2v = idx_k2[pl.ds(goff, _L)]
            joff = (jv & 7) << 4
            k0off = (k0v & 7) << 4
            k1off = (k1v & 7) << 4
            k2off = (k2v & 7) << 4

            mj = [plsc.load_gather(mj_buf, [slot, joff + r])
                  for r in range(_R)]
            ms = [
                plsc.load_gather(mk0_buf, [slot, k0off + r])
                + plsc.load_gather(mk1_buf, [slot, k1off + r])
                + plsc.load_gather(mk2_buf, [slot, k2off + r])
                for r in range(_R)
            ]

            def col(c):
                return jnp.full((_L,), c, dtype=jnp.int32)

            accp = None
            a = [None, None, None]
            b = [None, None, None]
            for c in range(_DPF):
                pv = plsc.load_gather(pf_buf, [grow, col(c)])
                if c < _R:
                    term = pv * mj[c]
                    accp = term if accp is None else accp + term
                elif c < (1 + _S) * _R:
                    r, s = divmod(c - _R, _S)
                    term = pv * mj[r]
                    a[s] = term if a[s] is None else a[s] + term
                else:
                    r, s = divmod(c - (1 + _S) * _R, _S)
                    term = pv * ms[r]
                    b[s] = term if b[s] is None else b[s] + term

            res = _ALPHA * accp + (_BETA * _BETA) * (
                a[0] * b[0] + a[1] * b[1] + a[2] * b[2])
            plsc.store_scatter(out_buf, [grow], res)
            return carry2

        lax.fori_loop(0, _NGC, group, 0)
        return carry

    cp_pf.wait()
    lax.fori_loop(0, _NCH, chunk_body, 0)

    pltpu.sync_copy(out_buf, out_hbm.at[pl.ds(base, _BPW)])


@jax.jit
def _mf_call(i, j, k0, k1, k2, pF128, M8):
    mesh = plsc.VectorSubcoreMesh(core_axis_name="c", subcore_axis_name="s")
    f = functools.partial(
        pl.kernel,
        mesh=mesh,
        out_type=jax.ShapeDtypeStruct((_BATCH,), jnp.float32),
        compiler_params=pltpu.CompilerParams(
            use_tc_tiling_on_sc=True, needs_layout_passes=False),
        scratch_types=[
            pltpu.VMEM((_BPW,), jnp.int32),
            pltpu.VMEM((_BPW,), jnp.int32),
            pltpu.VMEM((_BPW,), jnp.int32),
            pltpu.VMEM((_BPW,), jnp.int32),
            pltpu.VMEM((_BPW,), jnp.int32),
            pltpu.VMEM((_CHUNK,), jnp.int32),
            pltpu.VMEM((_CHUNK,), jnp.int32),
            pltpu.VMEM((_CHUNK,), jnp.int32),
            pltpu.VMEM((_CHUNK,), jnp.int32),
            pltpu.VMEM((_BPW, 128), jnp.float32),
            pltpu.VMEM((_CHUNK, 128), jnp.float32),
            pltpu.VMEM((_CHUNK, 128), jnp.float32),
            pltpu.VMEM((_CHUNK, 128), jnp.float32),
            pltpu.VMEM((_CHUNK, 128), jnp.float32),
            pltpu.VMEM((_BPW,), jnp.float32),
            pltpu.SemaphoreType.DMA,
            pltpu.SemaphoreType.DMA,
            pltpu.SemaphoreType.DMA,
            pltpu.SemaphoreType.DMA,
            pltpu.SemaphoreType.DMA,
        ],
    )(_mf_body)
    return f(i, j, k0, k1, k2, pF128, M8)


_PADBLK = 2000


def _pad_body(x_ref, o_ref):
    o_ref[...] = jnp.concatenate(
        [x_ref[...], jnp.zeros((_PADBLK, 128 - _DPF), jnp.float32)], axis=1)


def _pad_pf(pF):
    n = pF.shape[0]
    return pl.pallas_call(
        _pad_body,
        grid=(n // _PADBLK,),
        in_specs=[pl.BlockSpec((_PADBLK, _DPF), lambda g: (g, 0))],
        out_specs=pl.BlockSpec((_PADBLK, 128), lambda g: (g, 0)),
        out_shape=jax.ShapeDtypeStruct((n, 128), jnp.float32),
    )(pF)


def kernel(ijk, pF, M):
    i = ijk[:, 0]
    j = ijk[:, 1]
    k0 = ijk[:, 2]
    k1 = ijk[:, 3]
    k2 = ijk[:, 4]
    pF128 = _pad_pf(pF)
    M8 = M.reshape(-1, 128)
    return _mf_call(i, j, k0, k1, k2, pF128, M8)


# speedup vs baseline: 5.6793x; 1.3265x over previous
import functools

import jax
import jax.numpy as jnp
from jax import lax
from jax.experimental import pallas as pl
from jax.experimental.pallas import tpu as pltpu
from jax.experimental.pallas import tpu_sc as plsc

_ALPHA = 0.001
_BETA = 0.001
_S = 3
_R = 16
_DPF = _R * (1 + 2 * _S)
_BATCH = 16384
_NC, _NS, _L = 2, 16, 16
_NW = _NC * _NS
_BPW = _BATCH // _NW
_CHUNK = 64
_NCH = _BPW // _CHUNK
_NGC = _CHUNK // _L


def _mf_body(i_hbm, j_hbm, k0_hbm, k1_hbm, k2_hbm, pF_hbm, M8_hbm, out_hbm,
             idx_i,
             idx_j, idx_k0, idx_k1, idx_k2,
             gj, gk0, gk1, gk2,
             pf_buf, mj_buf, mk0_buf, mk1_buf, mk2_buf, out_buf,
             s0, s1, s2, s3, s4):
    wid = lax.axis_index("s") * _NC + lax.axis_index("c")
    base = wid * _BPW

    pltpu.sync_copy(i_hbm.at[pl.ds(base, _BPW)], idx_i)
    pltpu.sync_copy(j_hbm.at[pl.ds(base, _BPW)], idx_j)
    pltpu.sync_copy(k0_hbm.at[pl.ds(base, _BPW)], idx_k0)
    pltpu.sync_copy(k1_hbm.at[pl.ds(base, _BPW)], idx_k1)
    pltpu.sync_copy(k2_hbm.at[pl.ds(base, _BPW)], idx_k2)

    def pf_fetch(g, carry):
        goff = pl.multiple_of(g * _L, _L)
        ivec = idx_i[pl.ds(goff, _L)]
        for l in range(_L):
            iv = ivec[l]
            n = g * _L + l
            pltpu.make_async_copy(
                pF_hbm.at[pl.ds(iv, 1)], pf_buf.at[pl.ds(n, 1)], s0).start()
        return carry

    lax.fori_loop(0, _BPW // _L, pf_fetch, 0)
    pltpu.make_async_copy(pF_hbm.at[pl.ds(0, _BPW)], pf_buf, s0).wait()

    lane = lax.iota(jnp.int32, _L)

    def chunk_body(ch, carry):
        cb = ch * _CHUNK
        for idx_src, gdst in ((idx_j, gj), (idx_k0, gk0),
                              (idx_k1, gk1), (idx_k2, gk2)):
            for v in range(_CHUNK // _L):
                off = pl.multiple_of(cb + v * _L, _L)
                gdst[pl.ds(v * _L, _L)] = (
                    idx_src[pl.ds(off, _L)] >> 3)
        cps = [
            pltpu.async_copy(M8_hbm.at[gj], mj_buf, s1),
            pltpu.async_copy(M8_hbm.at[gk0], mk0_buf, s2),
            pltpu.async_copy(M8_hbm.at[gk1], mk1_buf, s3),
            pltpu.async_copy(M8_hbm.at[gk2], mk2_buf, s4),
        ]
        for cp in cps:
            cp.wait()

        def group(grp, carry2):
            slot = grp * _L + lane
            grow = cb + slot
            goff = pl.multiple_of(cb + grp * _L, _L)
            jv = idx_j[pl.ds(goff, _L)]
            k0v = idx_k0[pl.ds(goff, _L)]
            k1v = idx_k1[pl.ds(goff, _L)]
            k2v = idx_k2[pl.ds(goff, _L)]
            joff = (jv & 7) << 4
            k0off = (k0v & 7) << 4
            k1off = (k1v & 7) << 4
            k2off = (k2v & 7) << 4

            mj = [plsc.load_gather(mj_buf, [slot, joff + r])
                  for r in range(_R)]
            ms = [
                plsc.load_gather(mk0_buf, [slot, k0off + r])
                + plsc.load_gather(mk1_buf, [slot, k1off + r])
                + plsc.load_gather(mk2_buf, [slot, k2off + r])
                for r in range(_R)
            ]

            def col(c):
                return jnp.full((_L,), c, dtype=jnp.int32)

            accp = None
            a = [None, None, None]
            b = [None, None, None]
            for c in range(_DPF):
                pv = plsc.load_gather(pf_buf, [grow, col(c)])
                if c < _R:
                    term = pv * mj[c]
                    accp = term if accp is None else accp + term
                elif c < (1 + _S) * _R:
                    r, s = divmod(c - _R, _S)
                    term = pv * mj[r]
                    a[s] = term if a[s] is None else a[s] + term
                else:
                    r, s = divmod(c - (1 + _S) * _R, _S)
                    term = pv * ms[r]
                    b[s] = term if b[s] is None else b[s] + term

            res = _ALPHA * accp + (_BETA * _BETA) * (
                a[0] * b[0] + a[1] * b[1] + a[2] * b[2])
            plsc.store_scatter(out_buf, [grow], res)
            return carry2

        lax.fori_loop(0, _NGC, group, 0)
        return carry

    lax.fori_loop(0, _NCH, chunk_body, 0)

    pltpu.sync_copy(out_buf, out_hbm.at[pl.ds(base, _BPW)])


@jax.jit
def _mf_call(i, j, k0, k1, k2, pF, M8):
    mesh = plsc.VectorSubcoreMesh(core_axis_name="c", subcore_axis_name="s")
    f = functools.partial(
        pl.kernel,
        mesh=mesh,
        out_type=jax.ShapeDtypeStruct((_BATCH,), jnp.float32),
        compiler_params=pltpu.CompilerParams(
            use_tc_tiling_on_sc=True, needs_layout_passes=False),
        scratch_types=[
            pltpu.VMEM((_BPW,), jnp.int32),
            pltpu.VMEM((_BPW,), jnp.int32),
            pltpu.VMEM((_BPW,), jnp.int32),
            pltpu.VMEM((_BPW,), jnp.int32),
            pltpu.VMEM((_BPW,), jnp.int32),
            pltpu.VMEM((_CHUNK,), jnp.int32),
            pltpu.VMEM((_CHUNK,), jnp.int32),
            pltpu.VMEM((_CHUNK,), jnp.int32),
            pltpu.VMEM((_CHUNK,), jnp.int32),
            pltpu.VMEM((_BPW, _DPF), jnp.float32),
            pltpu.VMEM((_CHUNK, 128), jnp.float32),
            pltpu.VMEM((_CHUNK, 128), jnp.float32),
            pltpu.VMEM((_CHUNK, 128), jnp.float32),
            pltpu.VMEM((_CHUNK, 128), jnp.float32),
            pltpu.VMEM((_BPW,), jnp.float32),
            pltpu.SemaphoreType.DMA,
            pltpu.SemaphoreType.DMA,
            pltpu.SemaphoreType.DMA,
            pltpu.SemaphoreType.DMA,
            pltpu.SemaphoreType.DMA,
        ],
    )(_mf_body)
    return f(i, j, k0, k1, k2, pF, M8)


def kernel(ijk, pF, M):
    i = ijk[:, 0]
    j = ijk[:, 1]
    k0 = ijk[:, 2]
    k1 = ijk[:, 3]
    k2 = ijk[:, 4]
    M8 = M.reshape(-1, 128)
    return _mf_call(i, j, k0, k1, k2, pF, M8)


# speedup vs baseline: 5.9458x; 1.0469x over previous
import functools

import jax
import jax.numpy as jnp
from jax import lax
from jax.experimental import pallas as pl
from jax.experimental.pallas import tpu as pltpu
from jax.experimental.pallas import tpu_sc as plsc

_ALPHA = 0.001
_BETA = 0.001
_S = 3
_R = 16
_DPF = _R * (1 + 2 * _S)
_BATCH = 16384
_NC, _NS, _L = 2, 16, 16
_NW = _NC * _NS
_BPW = _BATCH // _NW
_NG = _BPW // _L


def _mf_body(ijkT_hbm, pF_hbm, M_hbm, out_hbm,
             idx_i, idx_j, idx_k0, idx_k1, idx_k2,
             pf_buf, m_buf, out_buf,
             s0, s1):
    wid = lax.axis_index("s") * _NC + lax.axis_index("c")
    base = wid * _BPW

    pltpu.sync_copy(ijkT_hbm.at[pl.ds(0 * _BATCH + base, _BPW)], idx_i)
    pltpu.sync_copy(ijkT_hbm.at[pl.ds(1 * _BATCH + base, _BPW)], idx_j)
    pltpu.sync_copy(ijkT_hbm.at[pl.ds(2 * _BATCH + base, _BPW)], idx_k0)
    pltpu.sync_copy(ijkT_hbm.at[pl.ds(3 * _BATCH + base, _BPW)], idx_k1)
    pltpu.sync_copy(ijkT_hbm.at[pl.ds(4 * _BATCH + base, _BPW)], idx_k2)

    lane = lax.iota(jnp.int32, _L)

    def fetch(g, carry):
        goff = pl.multiple_of(g * _L, _L)
        ivec = idx_i[pl.ds(goff, _L)]
        jvec = idx_j[pl.ds(goff, _L)]
        k0vec = idx_k0[pl.ds(goff, _L)]
        k1vec = idx_k1[pl.ds(goff, _L)]
        k2vec = idx_k2[pl.ds(goff, _L)]
        for l in range(_L):
            n = g * _L + l
            pltpu.make_async_copy(
                pF_hbm.at[pl.ds(ivec[l], 1)],
                pf_buf.at[pl.ds(n, 1)], s0).start()
            for q, vec in enumerate((jvec, k0vec, k1vec, k2vec)):
                pltpu.make_async_copy(
                    M_hbm.at[pl.ds(vec[l] * _R, _R)],
                    m_buf.at[pl.ds(n * 64 + q * 16, 16)],
                    s1).start()
        return carry

    lax.fori_loop(0, _NG, fetch, 0)

    pltpu.make_async_copy(pF_hbm.at[pl.ds(0, _BPW)], pf_buf, s0).wait()
    for q in range(2):
        pltpu.make_async_copy(
            out_hbm,
            m_buf.at[pl.ds(q * _BATCH, _BATCH)], s1).wait()

    def group(grp, carry):
        grow = grp * _L + lane
        mbase = grow << 6

        def col(c):
            return jnp.full((_L,), c, dtype=jnp.int32)

        mj = [plsc.load_gather(m_buf, [mbase + r]) for r in range(_R)]
        ms = [
            plsc.load_gather(m_buf, [mbase + (16 + r)])
            + plsc.load_gather(m_buf, [mbase + (32 + r)])
            + plsc.load_gather(m_buf, [mbase + (48 + r)])
            for r in range(_R)
        ]

        accp = None
        a = [None, None, None]
        b = [None, None, None]
        for c in range(_DPF):
            pv = plsc.load_gather(pf_buf, [grow, col(c)])
            if c < _R:
                term = pv * mj[c]
                accp = term if accp is None else accp + term
            elif c < (1 + _S) * _R:
                r, s = divmod(c - _R, _S)
                term = pv * mj[r]
                a[s] = term if a[s] is None else a[s] + term
            else:
                r, s = divmod(c - (1 + _S) * _R, _S)
                term = pv * ms[r]
                b[s] = term if b[s] is None else b[s] + term

        res = _ALPHA * accp + (_BETA * _BETA) * (
            a[0] * b[0] + a[1] * b[1] + a[2] * b[2])
        plsc.store_scatter(out_buf, [grow], res)
        return carry

    lax.fori_loop(0, _NG, group, 0)

    pltpu.sync_copy(out_buf, out_hbm.at[pl.ds(base, _BPW)])


@jax.jit
def _mf_call(ijkT_flat, pF, M):
    mesh = plsc.VectorSubcoreMesh(core_axis_name="c", subcore_axis_name="s")
    f = functools.partial(
        pl.kernel,
        mesh=mesh,
        out_type=jax.ShapeDtypeStruct((_BATCH,), jnp.float32),
        compiler_params=pltpu.CompilerParams(
            use_tc_tiling_on_sc=True, needs_layout_passes=False),
        scratch_types=[
            pltpu.VMEM((_BPW,), jnp.int32),
            pltpu.VMEM((_BPW,), jnp.int32),
            pltpu.VMEM((_BPW,), jnp.int32),
            pltpu.VMEM((_BPW,), jnp.int32),
            pltpu.VMEM((_BPW,), jnp.int32),
            pltpu.VMEM((_BPW, _DPF), jnp.float32),
            pltpu.VMEM((_BPW * 64,), jnp.float32),
            pltpu.VMEM((_BPW,), jnp.float32),
            pltpu.SemaphoreType.DMA,
            pltpu.SemaphoreType.DMA,
        ],
    )(_mf_body)
    return f(ijkT_flat, pF, M)


def kernel(ijk, pF, M):
    return _mf_call(ijk.T.reshape(-1), pF, M.reshape(-1))
